# baseline (speedup 1.0000x reference)
import functools

import jax
import jax.numpy as jnp
import numpy as np
from jax.experimental import pallas as pl

_N_GAUSS = 50
_CUTOFF = 5.0
_VARIANCE = 1.0
_LOG2 = float(np.log(2.0))


def _softplus(x):
    return jnp.maximum(x, 0.0) + jnp.log1p(jnp.exp(-jnp.abs(x)))


def _schnet_body(xd_ref, ep_ref, emb_ref, winit_ref, wf1_ref,
                 wf2_ref, bias_ref, wo1_ref, wo2_ref, m_ref, s_ref, out_ref,
                 *, P, B, FEAT):
    G = 64

    xd = xd_ref[0]
    ones3 = jnp.full((3, G), 1.0, jnp.float32)
    d2 = jnp.dot(xd * xd, ones3, preferred_element_type=jnp.float32,
                 precision=jax.lax.Precision.HIGHEST)
    mask = jnp.where(d2[:, :1] < _CUTOFF * _CUTOFF, 1.0, 0.0)

    d = jnp.sqrt(d2)
    g = jax.lax.broadcasted_iota(jnp.int32, (1, G), 1).astype(jnp.float32)
    centers = g * (_CUTOFF / (_N_GAUSS - 1))
    arg = d - centers
    rbf = jnp.exp(arg * arg * (-0.5 / _VARIANCE))

    def _bf(a):
        return a if a.dtype == jnp.bfloat16 else a.astype(jnp.bfloat16)

    def _bdot(a, b):
        return jnp.dot(_bf(a), _bf(b), preferred_element_type=jnp.float32)

    def _xdot(a, b):
        return jnp.dot(a, b, preferred_element_type=jnp.float32,
                       precision=jax.lax.Precision.HIGHEST)

    z1 = _bdot(rbf, wf1_ref[...])
    a1 = _softplus(z1 + bias_ref[0:1, :])
    filt = _bdot(a1, wf2_ref[...])
    filt = (filt + bias_ref[1:2, :]) * mask

    ep = ep_ref[0]
    vocab = jax.lax.broadcasted_iota(jnp.int32, (B, 64), 1)
    onehot = jnp.where(ep == vocab, 1.0, 0.0)
    feat = _xdot(onehot, emb_ref[...])

    M = m_ref[...]
    S = s_ref[...]

    fsum = _bdot(M, filt)

    for b in range(2):
        h = _xdot(feat, winit_ref[b])
        fb = filt[:, b * FEAT:(b + 1) * FEAT]
        hsum = _bdot(S, h)
        t1 = _bdot(M, fb * hsum)
        agg = t1 - h * fsum[:, b * FEAT:(b + 1) * FEAT]
        t = _xdot(agg, wo1_ref[b])
        t = _softplus(t + bias_ref[2 + 2 * b:3 + 2 * b, :FEAT])
        out = _xdot(t, wo2_ref[b])
        out = out + bias_ref[3 + 2 * b:4 + 2 * b, :FEAT]
        feat = feat + out

    out_ref[0] = feat


def kernel(in_features, embedding_property, emb_table, W_init, W_f1, b_f1,
           W_f2, b_f2, W_o1, b_o1, W_o2, b_o2):
    Fr, B, _ = in_features.shape
    N_EMB, FEAT = emb_table.shape
    G = 64

    pairs = np.asarray(
        [(i, j) for i in range(B) for j in range(i + 1, B)], dtype=np.int32)
    NP_REAL = pairs.shape[0]
    P = -(-NP_REAL // 256) * 256

    M_np = np.zeros((B, P), dtype=np.float32)
    M_np[pairs[:, 0], np.arange(NP_REAL)] = 1.0
    M_np[pairs[:, 1], np.arange(NP_REAL)] = 1.0
    M = jnp.asarray(M_np, dtype=jnp.bfloat16)
    S = jnp.asarray(M_np.T.copy(), dtype=jnp.bfloat16)

    x = in_features
    ia = np.zeros(P, dtype=np.int32)
    ib = np.zeros(P, dtype=np.int32)
    ia[:NP_REAL] = pairs[:, 0]
    ib[:NP_REAL] = pairs[:, 1]
    XD = (jnp.take(x, jnp.asarray(ia), axis=1)
          - jnp.take(x, jnp.asarray(ib), axis=1))
    ep3 = embedding_property.astype(jnp.int32).reshape(Fr, B, 1)
    emb_pad = jnp.pad(emb_table, ((0, 64 - N_EMB), (0, 0)))

    wf1p = jnp.pad(W_f1, ((0, 0), (0, G - _N_GAUSS), (0, 0)))
    W_f1c = jnp.concatenate([wf1p[0], wf1p[1]], axis=1)
    W_f2c = jnp.zeros((2 * FEAT, 2 * FEAT), jnp.float32)
    W_f2c = W_f2c.at[:FEAT, :FEAT].set(W_f2[0]).at[FEAT:, FEAT:].set(W_f2[1])
    W_f1c = W_f1c.astype(jnp.bfloat16)
    W_f2c = W_f2c.astype(jnp.bfloat16)

    def pad128(v):
        return jnp.pad(v, (0, 2 * FEAT - v.shape[0]))

    b_f2c = (jnp.concatenate([b_f2[0], b_f2[1]])
             - _LOG2 * jnp.sum(W_f2c, axis=0))
    bias_pack = jnp.stack([
        jnp.concatenate([b_f1[0], b_f1[1]]),
        b_f2c,
        pad128(b_o1[0]), pad128(b_o2[0] - _LOG2 * jnp.sum(W_o2[0], axis=0)),
        pad128(b_o1[1]), pad128(b_o2[1] - _LOG2 * jnp.sum(W_o2[1], axis=0)),
        jnp.zeros(2 * FEAT), jnp.zeros(2 * FEAT),
    ])

    body = functools.partial(_schnet_body, P=P, B=B, FEAT=FEAT)
    out = pl.pallas_call(
        body,
        grid=(Fr,),
        in_specs=[
            pl.BlockSpec((1, P, 3), lambda f: (f, 0, 0)),
            pl.BlockSpec((1, B, 1), lambda f: (f, 0, 0)),
            pl.BlockSpec((64, FEAT), lambda f: (0, 0)),
            pl.BlockSpec((2, FEAT, FEAT), lambda f: (0, 0, 0)),
            pl.BlockSpec((G, 2 * FEAT), lambda f: (0, 0)),
            pl.BlockSpec((2 * FEAT, 2 * FEAT), lambda f: (0, 0)),
            pl.BlockSpec((8, 2 * FEAT), lambda f: (0, 0)),
            pl.BlockSpec((2, FEAT, FEAT), lambda f: (0, 0, 0)),
            pl.BlockSpec((2, FEAT, FEAT), lambda f: (0, 0, 0)),
            pl.BlockSpec((B, P), lambda f: (0, 0)),
            pl.BlockSpec((P, B), lambda f: (0, 0)),
        ],
        out_specs=pl.BlockSpec((1, B, FEAT), lambda f: (f, 0, 0)),
        out_shape=jax.ShapeDtypeStruct((Fr, B, FEAT), jnp.float32),
    )(XD, ep3, emb_pad, W_init, W_f1c, W_f2c, bias_pack, W_o1, W_o2, M, S)
    return out


# speedup vs baseline: 1.1901x; 1.1901x over previous
import functools

import jax
import jax.numpy as jnp
import numpy as np
from jax.experimental import pallas as pl

_N_GAUSS = 50
_CUTOFF = 5.0
_VARIANCE = 1.0
_LOG2 = float(np.log(2.0))


def _softplus(x):
    return jnp.maximum(x, 0.0) + jnp.log1p(jnp.exp(-jnp.abs(x)))


def _schnet_body(xd_ref, ep_ref, emb_ref, winit_ref, wf1_ref,
                 wf2_ref, bias_ref, wo1_ref, wo2_ref, m_ref, s_ref, out_ref,
                 *, P, B, FEAT):
    G = 64

    xd = xd_ref[0]
    x0 = xd[:, 0:1]
    x1 = xd[:, 1:2]
    x2 = xd[:, 2:3]
    dc = jnp.sqrt((x0 * x0 + x1 * x1) + x2 * x2)
    mask = jnp.where(dc < _CUTOFF, 1.0, 0.0)

    d = jax.lax.broadcast_in_dim(dc, (P, G), (0, 1))
    g = jax.lax.broadcasted_iota(jnp.int32, (1, G), 1).astype(jnp.float32)
    centers = g * (_CUTOFF / (_N_GAUSS - 1))
    arg = d - centers
    rbf = jnp.exp(arg * arg * (-0.5 / _VARIANCE))

    def _bf(a):
        return a if a.dtype == jnp.bfloat16 else a.astype(jnp.bfloat16)

    def _bdot(a, b):
        return jnp.dot(_bf(a), _bf(b), preferred_element_type=jnp.float32)

    def _xdot(a, b):
        return jnp.dot(a, b, preferred_element_type=jnp.float32,
                       precision=jax.lax.Precision.HIGHEST)

    z1 = _bdot(rbf, wf1_ref[...])
    a1 = _softplus(z1 + bias_ref[0:1, :])
    filt = _bdot(a1, wf2_ref[...])
    filt = (filt + bias_ref[1:2, :]) * mask

    ep = ep_ref[0]
    vocab = jax.lax.broadcasted_iota(jnp.int32, (B, 64), 1)
    onehot = jnp.where(ep == vocab, 1.0, 0.0)
    feat = _xdot(onehot, emb_ref[...])

    M = m_ref[...]
    S = s_ref[...]

    fsum = _bdot(M, filt)

    for b in range(2):
        h = _xdot(feat, winit_ref[b])
        fb = filt[:, b * FEAT:(b + 1) * FEAT]
        hsum = _bdot(S, h)
        t1 = _bdot(M, fb * hsum)
        agg = t1 - h * fsum[:, b * FEAT:(b + 1) * FEAT]
        t = _xdot(agg, wo1_ref[b])
        t = _softplus(t + bias_ref[2 + 2 * b:3 + 2 * b, :FEAT])
        out = _xdot(t, wo2_ref[b])
        out = out + bias_ref[3 + 2 * b:4 + 2 * b, :FEAT]
        feat = feat + out

    out_ref[0] = feat


def kernel(in_features, embedding_property, emb_table, W_init, W_f1, b_f1,
           W_f2, b_f2, W_o1, b_o1, W_o2, b_o2):
    Fr, B, _ = in_features.shape
    N_EMB, FEAT = emb_table.shape
    G = 64

    pairs = np.asarray(
        [(i, j) for i in range(B) for j in range(i + 1, B)], dtype=np.int32)
    NP_REAL = pairs.shape[0]
    P = -(-NP_REAL // 256) * 256

    M_np = np.zeros((B, P), dtype=np.float32)
    M_np[pairs[:, 0], np.arange(NP_REAL)] = 1.0
    M_np[pairs[:, 1], np.arange(NP_REAL)] = 1.0
    M = jnp.asarray(M_np, dtype=jnp.bfloat16)
    S = jnp.asarray(M_np.T.copy(), dtype=jnp.bfloat16)

    x = in_features
    ia = np.zeros(P, dtype=np.int32)
    ib = np.zeros(P, dtype=np.int32)
    ia[:NP_REAL] = pairs[:, 0]
    ib[:NP_REAL] = pairs[:, 1]
    XD = (jnp.take(x, jnp.asarray(ia), axis=1)
          - jnp.take(x, jnp.asarray(ib), axis=1))
    ep3 = embedding_property.astype(jnp.int32).reshape(Fr, B, 1)
    emb_pad = jnp.pad(emb_table, ((0, 64 - N_EMB), (0, 0)))

    wf1p = jnp.pad(W_f1, ((0, 0), (0, G - _N_GAUSS), (0, 0)))
    W_f1c = jnp.concatenate([wf1p[0], wf1p[1]], axis=1)
    W_f2c = jnp.zeros((2 * FEAT, 2 * FEAT), jnp.float32)
    W_f2c = W_f2c.at[:FEAT, :FEAT].set(W_f2[0]).at[FEAT:, FEAT:].set(W_f2[1])

    def pad128(v):
        return jnp.pad(v, (0, 2 * FEAT - v.shape[0]))

    b_f2c = (jnp.concatenate([b_f2[0], b_f2[1]])
             - _LOG2 * jnp.sum(W_f2c, axis=0))
    W_f1c = W_f1c.astype(jnp.bfloat16)
    W_f2c = W_f2c.astype(jnp.bfloat16)
    bias_pack = jnp.stack([
        jnp.concatenate([b_f1[0], b_f1[1]]),
        b_f2c,
        pad128(b_o1[0]), pad128(b_o2[0] - _LOG2 * jnp.sum(W_o2[0], axis=0)),
        pad128(b_o1[1]), pad128(b_o2[1] - _LOG2 * jnp.sum(W_o2[1], axis=0)),
        jnp.zeros(2 * FEAT), jnp.zeros(2 * FEAT),
    ])

    body = functools.partial(_schnet_body, P=P, B=B, FEAT=FEAT)
    out = pl.pallas_call(
        body,
        grid=(Fr,),
        in_specs=[
            pl.BlockSpec((1, P, 3), lambda f: (f, 0, 0)),
            pl.BlockSpec((1, B, 1), lambda f: (f, 0, 0)),
            pl.BlockSpec((64, FEAT), lambda f: (0, 0)),
            pl.BlockSpec((2, FEAT, FEAT), lambda f: (0, 0, 0)),
            pl.BlockSpec((G, 2 * FEAT), lambda f: (0, 0)),
            pl.BlockSpec((2 * FEAT, 2 * FEAT), lambda f: (0, 0)),
            pl.BlockSpec((8, 2 * FEAT), lambda f: (0, 0)),
            pl.BlockSpec((2, FEAT, FEAT), lambda f: (0, 0, 0)),
            pl.BlockSpec((2, FEAT, FEAT), lambda f: (0, 0, 0)),
            pl.BlockSpec((B, P), lambda f: (0, 0)),
            pl.BlockSpec((P, B), lambda f: (0, 0)),
        ],
        out_specs=pl.BlockSpec((1, B, FEAT), lambda f: (f, 0, 0)),
        out_shape=jax.ShapeDtypeStruct((Fr, B, FEAT), jnp.float32),
    )(XD, ep3, emb_pad, W_init, W_f1c, W_f2c, bias_pack, W_o1, W_o2, M, S)
    return out


# speedup vs baseline: 1.2868x; 1.0812x over previous
import functools

import jax
import jax.numpy as jnp
import numpy as np
from jax.experimental import pallas as pl

_N_GAUSS = 50
_CUTOFF = 5.0
_VARIANCE = 1.0
_LOG2 = float(np.log(2.0))


def _softplus(x):
    return jnp.maximum(x, 0.0) + jnp.log1p(jnp.exp(-jnp.abs(x)))


def _schnet_body(xd_ref, ep_ref, emb_ref, winit_ref, wf1_ref,
                 wf2_ref, bias_ref, wo1_ref, wo2_ref, m_ref, s_ref, out_ref,
                 *, P, B, FEAT, FPB):
    G = 64

    def _bf(a):
        return a if a.dtype == jnp.bfloat16 else a.astype(jnp.bfloat16)

    def _bdot(a, b):
        return jnp.dot(_bf(a), _bf(b), preferred_element_type=jnp.float32)

    def _xdot(a, b):
        return jnp.dot(a, b, preferred_element_type=jnp.float32,
                       precision=jax.lax.Precision.HIGHEST)

    M = m_ref[...]
    S = s_ref[...]
    g = jax.lax.broadcasted_iota(jnp.int32, (1, G), 1).astype(jnp.float32)
    centers = g * (_CUTOFF / (_N_GAUSS - 1))
    vocab = jax.lax.broadcasted_iota(jnp.int32, (B, 64), 1)

    for ff in range(FPB):
        xd = xd_ref[ff]
        x0 = xd[:, 0:1]
        x1 = xd[:, 1:2]
        x2 = xd[:, 2:3]
        dc = jnp.sqrt((x0 * x0 + x1 * x1) + x2 * x2)
        mask = jnp.where(dc < _CUTOFF, 1.0, 0.0)

        d = jax.lax.broadcast_in_dim(dc, (P, G), (0, 1))
        arg = d - centers
        rbf = jnp.exp(arg * arg * (-0.5 / _VARIANCE))

        z1 = _bdot(rbf, wf1_ref[...])
        a1 = _softplus(z1 + bias_ref[0:1, :])
        filt = _bdot(a1, wf2_ref[...])
        filt = (filt + bias_ref[1:2, :]) * mask

        ep = ep_ref[ff]
        onehot = jnp.where(ep == vocab, 1.0, 0.0)
        feat = _xdot(onehot, emb_ref[...])

        fsum = _bdot(M, filt)

        for b in range(2):
            h = _xdot(feat, winit_ref[b])
            fb = filt[:, b * FEAT:(b + 1) * FEAT]
            hsum = _bdot(S, h)
            t1 = _bdot(M, fb * hsum)
            agg = t1 - h * fsum[:, b * FEAT:(b + 1) * FEAT]
            t = _xdot(agg, wo1_ref[b])
            t = _softplus(t + bias_ref[2 + 2 * b:3 + 2 * b, :FEAT])
            out = _xdot(t, wo2_ref[b])
            out = out + bias_ref[3 + 2 * b:4 + 2 * b, :FEAT]
            feat = feat + out

        out_ref[ff] = feat


def kernel(in_features, embedding_property, emb_table, W_init, W_f1, b_f1,
           W_f2, b_f2, W_o1, b_o1, W_o2, b_o2):
    Fr, B, _ = in_features.shape
    N_EMB, FEAT = emb_table.shape
    G = 64

    pairs = np.asarray(
        [(i, j) for i in range(B) for j in range(i + 1, B)], dtype=np.int32)
    NP_REAL = pairs.shape[0]
    P = -(-NP_REAL // 256) * 256

    M_np = np.zeros((B, P), dtype=np.float32)
    M_np[pairs[:, 0], np.arange(NP_REAL)] = 1.0
    M_np[pairs[:, 1], np.arange(NP_REAL)] = 1.0
    M = jnp.asarray(M_np, dtype=jnp.bfloat16)
    S = jnp.asarray(M_np.T.copy(), dtype=jnp.bfloat16)

    x = in_features
    ia = np.zeros(P, dtype=np.int32)
    ib = np.zeros(P, dtype=np.int32)
    ia[:NP_REAL] = pairs[:, 0]
    ib[:NP_REAL] = pairs[:, 1]
    XD = (jnp.take(x, jnp.asarray(ia), axis=1)
          - jnp.take(x, jnp.asarray(ib), axis=1))
    ep3 = embedding_property.astype(jnp.int32).reshape(Fr, B, 1)
    emb_pad = jnp.pad(emb_table, ((0, 64 - N_EMB), (0, 0)))

    wf1p = jnp.pad(W_f1, ((0, 0), (0, G - _N_GAUSS), (0, 0)))
    W_f1c = jnp.concatenate([wf1p[0], wf1p[1]], axis=1)
    W_f2c = jnp.zeros((2 * FEAT, 2 * FEAT), jnp.float32)
    W_f2c = W_f2c.at[:FEAT, :FEAT].set(W_f2[0]).at[FEAT:, FEAT:].set(W_f2[1])

    def pad128(v):
        return jnp.pad(v, (0, 2 * FEAT - v.shape[0]))

    b_f2c = (jnp.concatenate([b_f2[0], b_f2[1]])
             - _LOG2 * jnp.sum(W_f2c, axis=0))
    W_f1c = W_f1c.astype(jnp.bfloat16)
    W_f2c = W_f2c.astype(jnp.bfloat16)
    bias_pack = jnp.stack([
        jnp.concatenate([b_f1[0], b_f1[1]]),
        b_f2c,
        pad128(b_o1[0]), pad128(b_o2[0] - _LOG2 * jnp.sum(W_o2[0], axis=0)),
        pad128(b_o1[1]), pad128(b_o2[1] - _LOG2 * jnp.sum(W_o2[1], axis=0)),
        jnp.zeros(2 * FEAT), jnp.zeros(2 * FEAT),
    ])

    FPB = 2
    body = functools.partial(_schnet_body, P=P, B=B, FEAT=FEAT, FPB=FPB)
    out = pl.pallas_call(
        body,
        grid=(Fr // FPB,),
        in_specs=[
            pl.BlockSpec((FPB, P, 3), lambda f: (f, 0, 0)),
            pl.BlockSpec((FPB, B, 1), lambda f: (f, 0, 0)),
            pl.BlockSpec((64, FEAT), lambda f: (0, 0)),
            pl.BlockSpec((2, FEAT, FEAT), lambda f: (0, 0, 0)),
            pl.BlockSpec((G, 2 * FEAT), lambda f: (0, 0)),
            pl.BlockSpec((2 * FEAT, 2 * FEAT), lambda f: (0, 0)),
            pl.BlockSpec((8, 2 * FEAT), lambda f: (0, 0)),
            pl.BlockSpec((2, FEAT, FEAT), lambda f: (0, 0, 0)),
            pl.BlockSpec((2, FEAT, FEAT), lambda f: (0, 0, 0)),
            pl.BlockSpec((B, P), lambda f: (0, 0)),
            pl.BlockSpec((P, B), lambda f: (0, 0)),
        ],
        out_specs=pl.BlockSpec((FPB, B, FEAT), lambda f: (f, 0, 0)),
        out_shape=jax.ShapeDtypeStruct((Fr, B, FEAT), jnp.float32),
    )(XD, ep3, emb_pad, W_init, W_f1c, W_f2c, bias_pack, W_o1, W_o2, M, S)
    return out


# speedup vs baseline: 1.3020x; 1.0119x over previous
import functools

import jax
import jax.numpy as jnp
import numpy as np
from jax.experimental import pallas as pl

_N_GAUSS = 50
_CUTOFF = 5.0
_VARIANCE = 1.0
_LOG2 = float(np.log(2.0))


def _softplus(x):
    return jnp.maximum(x, 0.0) + jnp.log1p(jnp.exp(-jnp.abs(x)))


def _schnet_body(xd_ref, ep_ref, emb_ref, winit_ref, wf1_ref,
                 wf2_ref, bias_ref, wo1_ref, wo2_ref, m_ref, s_ref, out_ref,
                 *, P, B, FEAT, FPB):
    G = 64

    def _bf(a):
        return a if a.dtype == jnp.bfloat16 else a.astype(jnp.bfloat16)

    def _bdot(a, b):
        return jnp.dot(_bf(a), _bf(b), preferred_element_type=jnp.float32)

    def _split(w):
        hi = w.astype(jnp.bfloat16)
        lo = (w - hi.astype(jnp.float32)).astype(jnp.bfloat16)
        return hi, lo

    def _pdot(a, bhl):
        bh, bl = bhl
        ah, al = _split(a)
        return (jnp.dot(ah, bh, preferred_element_type=jnp.float32)
                + jnp.dot(ah, bl, preferred_element_type=jnp.float32)
                + jnp.dot(al, bh, preferred_element_type=jnp.float32))

    M = m_ref[...]
    S = s_ref[...]
    g = jax.lax.broadcasted_iota(jnp.int32, (1, G), 1).astype(jnp.float32)
    centers = g * (_CUTOFF / (_N_GAUSS - 1))
    vocab = jax.lax.broadcasted_iota(jnp.int32, (B, 64), 1)

    emb_hl = _split(emb_ref[...])
    winit_hl = [_split(winit_ref[b]) for b in range(2)]
    wo1_hl = [_split(wo1_ref[b]) for b in range(2)]
    wo2_hl = [_split(wo2_ref[b]) for b in range(2)]

    for ff in range(FPB):
        xd = xd_ref[ff]
        x0 = xd[:, 0:1]
        x1 = xd[:, 1:2]
        x2 = xd[:, 2:3]
        dc = jnp.sqrt((x0 * x0 + x1 * x1) + x2 * x2)
        mask = jnp.where(dc < _CUTOFF, 1.0, 0.0)

        d = jax.lax.broadcast_in_dim(dc, (P, G), (0, 1))
        arg = d - centers
        rbf = jnp.exp(arg * arg * (-0.5 / _VARIANCE))

        z1 = _bdot(rbf, wf1_ref[...])
        a1 = _softplus(z1 + bias_ref[0:1, :])
        filt = _bdot(a1, wf2_ref[...])
        filt = (filt + bias_ref[1:2, :]) * mask

        ep = ep_ref[ff]
        onehot = jnp.where(ep == vocab, 1.0, 0.0)
        oh = onehot.astype(jnp.bfloat16)
        feat = (jnp.dot(oh, emb_hl[0], preferred_element_type=jnp.float32)
                + jnp.dot(oh, emb_hl[1], preferred_element_type=jnp.float32))

        fsum = _bdot(M, filt)

        for b in range(2):
            h = _pdot(feat, winit_hl[b])
            fb = filt[:, b * FEAT:(b + 1) * FEAT]
            hsum = _bdot(S, h)
            t1 = _bdot(M, fb * hsum)
            agg = t1 - h * fsum[:, b * FEAT:(b + 1) * FEAT]
            t = _pdot(agg, wo1_hl[b])
            t = _softplus(t + bias_ref[2 + 2 * b:3 + 2 * b, :FEAT])
            out = _pdot(t, wo2_hl[b])
            out = out + bias_ref[3 + 2 * b:4 + 2 * b, :FEAT]
            feat = feat + out

        out_ref[ff] = feat


def kernel(in_features, embedding_property, emb_table, W_init, W_f1, b_f1,
           W_f2, b_f2, W_o1, b_o1, W_o2, b_o2):
    Fr, B, _ = in_features.shape
    N_EMB, FEAT = emb_table.shape
    G = 64

    pairs = np.asarray(
        [(i, j) for i in range(B) for j in range(i + 1, B)], dtype=np.int32)
    NP_REAL = pairs.shape[0]
    P = -(-NP_REAL // 256) * 256

    M_np = np.zeros((B, P), dtype=np.float32)
    M_np[pairs[:, 0], np.arange(NP_REAL)] = 1.0
    M_np[pairs[:, 1], np.arange(NP_REAL)] = 1.0
    M = jnp.asarray(M_np, dtype=jnp.bfloat16)
    S = jnp.asarray(M_np.T.copy(), dtype=jnp.bfloat16)

    x = in_features
    ia = np.zeros(P, dtype=np.int32)
    ib = np.zeros(P, dtype=np.int32)
    ia[:NP_REAL] = pairs[:, 0]
    ib[:NP_REAL] = pairs[:, 1]
    XD = (jnp.take(x, jnp.asarray(ia), axis=1)
          - jnp.take(x, jnp.asarray(ib), axis=1))
    ep3 = embedding_property.astype(jnp.int32).reshape(Fr, B, 1)
    emb_pad = jnp.pad(emb_table, ((0, 64 - N_EMB), (0, 0)))

    wf1p = jnp.pad(W_f1, ((0, 0), (0, G - _N_GAUSS), (0, 0)))
    W_f1c = jnp.concatenate([wf1p[0], wf1p[1]], axis=1)
    W_f2c = jnp.zeros((2 * FEAT, 2 * FEAT), jnp.float32)
    W_f2c = W_f2c.at[:FEAT, :FEAT].set(W_f2[0]).at[FEAT:, FEAT:].set(W_f2[1])

    def pad128(v):
        return jnp.pad(v, (0, 2 * FEAT - v.shape[0]))

    b_f2c = (jnp.concatenate([b_f2[0], b_f2[1]])
             - _LOG2 * jnp.sum(W_f2c, axis=0))
    W_f1c = W_f1c.astype(jnp.bfloat16)
    W_f2c = W_f2c.astype(jnp.bfloat16)
    bias_pack = jnp.stack([
        jnp.concatenate([b_f1[0], b_f1[1]]),
        b_f2c,
        pad128(b_o1[0]), pad128(b_o2[0] - _LOG2 * jnp.sum(W_o2[0], axis=0)),
        pad128(b_o1[1]), pad128(b_o2[1] - _LOG2 * jnp.sum(W_o2[1], axis=0)),
        jnp.zeros(2 * FEAT), jnp.zeros(2 * FEAT),
    ])

    FPB = 2
    body = functools.partial(_schnet_body, P=P, B=B, FEAT=FEAT, FPB=FPB)
    out = pl.pallas_call(
        body,
        grid=(Fr // FPB,),
        in_specs=[
            pl.BlockSpec((FPB, P, 3), lambda f: (f, 0, 0)),
            pl.BlockSpec((FPB, B, 1), lambda f: (f, 0, 0)),
            pl.BlockSpec((64, FEAT), lambda f: (0, 0)),
            pl.BlockSpec((2, FEAT, FEAT), lambda f: (0, 0, 0)),
            pl.BlockSpec((G, 2 * FEAT), lambda f: (0, 0)),
            pl.BlockSpec((2 * FEAT, 2 * FEAT), lambda f: (0, 0)),
            pl.BlockSpec((8, 2 * FEAT), lambda f: (0, 0)),
            pl.BlockSpec((2, FEAT, FEAT), lambda f: (0, 0, 0)),
            pl.BlockSpec((2, FEAT, FEAT), lambda f: (0, 0, 0)),
            pl.BlockSpec((B, P), lambda f: (0, 0)),
            pl.BlockSpec((P, B), lambda f: (0, 0)),
        ],
        out_specs=pl.BlockSpec((FPB, B, FEAT), lambda f: (f, 0, 0)),
        out_shape=jax.ShapeDtypeStruct((Fr, B, FEAT), jnp.float32),
    )(XD, ep3, emb_pad, W_init, W_f1c, W_f2c, bias_pack, W_o1, W_o2, M, S)
    return out


# speedup vs baseline: 1.3337x; 1.0243x over previous
import functools

import jax
import jax.numpy as jnp
import numpy as np
from jax.experimental import pallas as pl

_N_GAUSS = 50
_CUTOFF = 5.0
_VARIANCE = 1.0
_LOG2 = float(np.log(2.0))


def _softplus(x):
    return jnp.maximum(x, 0.0) + jnp.log1p(jnp.exp(-jnp.abs(x)))


def _schnet_body(xd_ref, ep_ref, emb_ref, winit_ref, wf1_ref,
                 wf2_ref, bias_ref, wo1_ref, wo2_ref, m_ref, s_ref, out_ref,
                 *, P, B, FEAT, FPB):
    G = 64

    def _bf(a):
        return a if a.dtype == jnp.bfloat16 else a.astype(jnp.bfloat16)

    def _bdot(a, b):
        return jnp.dot(_bf(a), _bf(b), preferred_element_type=jnp.float32)

    def _split(w):
        hi = w.astype(jnp.bfloat16)
        lo = (w - hi.astype(jnp.float32)).astype(jnp.bfloat16)
        return hi, lo

    def _pdot(a, bhl):
        bh, bl = bhl
        ah, al = _split(a)
        return (jnp.dot(ah, bh, preferred_element_type=jnp.float32)
                + jnp.dot(ah, bl, preferred_element_type=jnp.float32)
                + jnp.dot(al, bh, preferred_element_type=jnp.float32))

    M = m_ref[...]
    S = s_ref[...]
    g = jax.lax.broadcasted_iota(jnp.int32, (1, G), 1).astype(jnp.float32)
    centers = g * (_CUTOFF / (_N_GAUSS - 1))
    vocab = jax.lax.broadcasted_iota(jnp.int32, (B, 64), 1)

    emb_hl = _split(emb_ref[...])
    winit_hl = [_split(winit_ref[b]) for b in range(2)]
    wo1_hl = [_split(wo1_ref[b]) for b in range(2)]
    wo2_hl = [_split(wo2_ref[b]) for b in range(2)]

    for ff in range(FPB):
        xd = xd_ref[ff]
        x0 = xd[:, 0:1]
        x1 = xd[:, 1:2]
        x2 = xd[:, 2:3]
        dc = jnp.sqrt((x0 * x0 + x1 * x1) + x2 * x2)
        mask = jnp.where(dc < _CUTOFF, 1.0, 0.0)

        d = jax.lax.broadcast_in_dim(dc, (P, G), (0, 1))
        arg = d - centers
        rbf = jnp.exp(arg * arg * (-0.5 / _VARIANCE))

        z1 = _bdot(rbf, wf1_ref[...])
        a1 = _softplus(z1 + bias_ref[0:1, :])
        filt = _bdot(a1, wf2_ref[...])
        filt = (filt + bias_ref[1:2, :]) * mask

        ep = ep_ref[ff]
        onehot = jnp.where(ep == vocab, 1.0, 0.0)
        oh = onehot.astype(jnp.bfloat16)
        feat = (jnp.dot(oh, emb_hl[0], preferred_element_type=jnp.float32)
                + jnp.dot(oh, emb_hl[1], preferred_element_type=jnp.float32))

        fsum = _bdot(M, filt)

        for b in range(2):
            h = _pdot(feat, winit_hl[b])
            fb = filt[:, b * FEAT:(b + 1) * FEAT]
            hsum = _bdot(S, h)
            t1 = _bdot(M, fb * hsum)
            agg = t1 - h * fsum[:, b * FEAT:(b + 1) * FEAT]
            t = _pdot(agg, wo1_hl[b])
            t = _softplus(t + bias_ref[2 + 2 * b:3 + 2 * b, :FEAT])
            out = _pdot(t, wo2_hl[b])
            out = out + bias_ref[3 + 2 * b:4 + 2 * b, :FEAT]
            feat = feat + out

        out_ref[ff] = feat


def kernel(in_features, embedding_property, emb_table, W_init, W_f1, b_f1,
           W_f2, b_f2, W_o1, b_o1, W_o2, b_o2):
    Fr, B, _ = in_features.shape
    N_EMB, FEAT = emb_table.shape
    G = 64

    pairs = np.asarray(
        [(i, j) for i in range(B) for j in range(i + 1, B)], dtype=np.int32)
    NP_REAL = pairs.shape[0]
    P = -(-NP_REAL // 256) * 256

    M_np = np.zeros((B, P), dtype=np.float32)
    M_np[pairs[:, 0], np.arange(NP_REAL)] = 1.0
    M_np[pairs[:, 1], np.arange(NP_REAL)] = 1.0
    M = jnp.asarray(M_np, dtype=jnp.bfloat16)
    S = jnp.asarray(M_np.T.copy(), dtype=jnp.bfloat16)

    x = in_features
    ia = np.zeros(P, dtype=np.int32)
    ib = np.zeros(P, dtype=np.int32)
    ia[:NP_REAL] = pairs[:, 0]
    ib[:NP_REAL] = pairs[:, 1]
    XD = (jnp.take(x, jnp.asarray(ia), axis=1)
          - jnp.take(x, jnp.asarray(ib), axis=1))
    ep3 = embedding_property.astype(jnp.int32).reshape(Fr, B, 1)
    emb_pad = jnp.pad(emb_table, ((0, 64 - N_EMB), (0, 0)))

    wf1p = jnp.pad(W_f1, ((0, 0), (0, G - _N_GAUSS), (0, 0)))
    W_f1c = jnp.concatenate([wf1p[0], wf1p[1]], axis=1)
    W_f2c = jnp.zeros((2 * FEAT, 2 * FEAT), jnp.float32)
    W_f2c = W_f2c.at[:FEAT, :FEAT].set(W_f2[0]).at[FEAT:, FEAT:].set(W_f2[1])

    def pad128(v):
        return jnp.pad(v, (0, 2 * FEAT - v.shape[0]))

    b_f2c = (jnp.concatenate([b_f2[0], b_f2[1]])
             - _LOG2 * jnp.sum(W_f2c, axis=0))
    W_f1c = W_f1c.astype(jnp.bfloat16)
    W_f2c = W_f2c.astype(jnp.bfloat16)
    bias_pack = jnp.stack([
        jnp.concatenate([b_f1[0], b_f1[1]]),
        b_f2c,
        pad128(b_o1[0]), pad128(b_o2[0] - _LOG2 * jnp.sum(W_o2[0], axis=0)),
        pad128(b_o1[1]), pad128(b_o2[1] - _LOG2 * jnp.sum(W_o2[1], axis=0)),
        jnp.zeros(2 * FEAT), jnp.zeros(2 * FEAT),
    ])

    FPB = 4
    body = functools.partial(_schnet_body, P=P, B=B, FEAT=FEAT, FPB=FPB)
    out = pl.pallas_call(
        body,
        grid=(Fr // FPB,),
        in_specs=[
            pl.BlockSpec((FPB, P, 3), lambda f: (f, 0, 0)),
            pl.BlockSpec((FPB, B, 1), lambda f: (f, 0, 0)),
            pl.BlockSpec((64, FEAT), lambda f: (0, 0)),
            pl.BlockSpec((2, FEAT, FEAT), lambda f: (0, 0, 0)),
            pl.BlockSpec((G, 2 * FEAT), lambda f: (0, 0)),
            pl.BlockSpec((2 * FEAT, 2 * FEAT), lambda f: (0, 0)),
            pl.BlockSpec((8, 2 * FEAT), lambda f: (0, 0)),
            pl.BlockSpec((2, FEAT, FEAT), lambda f: (0, 0, 0)),
            pl.BlockSpec((2, FEAT, FEAT), lambda f: (0, 0, 0)),
            pl.BlockSpec((B, P), lambda f: (0, 0)),
            pl.BlockSpec((P, B), lambda f: (0, 0)),
        ],
        out_specs=pl.BlockSpec((FPB, B, FEAT), lambda f: (f, 0, 0)),
        out_shape=jax.ShapeDtypeStruct((Fr, B, FEAT), jnp.float32),
    )(XD, ep3, emb_pad, W_init, W_f1c, W_f2c, bias_pack, W_o1, W_o2, M, S)
    return out


# speedup vs baseline: 1.3472x; 1.0101x over previous
import functools

import jax
import jax.numpy as jnp
import numpy as np
from jax.experimental import pallas as pl

_N_GAUSS = 50
_CUTOFF = 5.0
_VARIANCE = 1.0
_LOG2 = float(np.log(2.0))


def _softplus(x):
    return jnp.maximum(x, 0.0) + jnp.log1p(jnp.exp(-jnp.abs(x)))


def _schnet_body(xd_ref, ep_ref, emb_ref, winit_ref, wf1_ref,
                 wf2_ref, bias_ref, wo1_ref, wo2_ref, m_ref, s_ref, out_ref,
                 *, P, B, FEAT, FPB):
    G = 64

    def _bf(a):
        return a if a.dtype == jnp.bfloat16 else a.astype(jnp.bfloat16)

    def _bdot(a, b):
        return jnp.dot(_bf(a), _bf(b), preferred_element_type=jnp.float32)

    def _split(w):
        hi = w.astype(jnp.bfloat16)
        lo = (w - hi.astype(jnp.float32)).astype(jnp.bfloat16)
        return hi, lo

    def _pdot(a, bhl):
        bh, bl = bhl
        ah, al = _split(a)
        return (jnp.dot(ah, bh, preferred_element_type=jnp.float32)
                + jnp.dot(ah, bl, preferred_element_type=jnp.float32)
                + jnp.dot(al, bh, preferred_element_type=jnp.float32))

    M = m_ref[...]
    S = s_ref[...]
    g = jax.lax.broadcasted_iota(jnp.int32, (1, G), 1).astype(jnp.float32)
    centers = g * (_CUTOFF / (_N_GAUSS - 1))
    vocab = jax.lax.broadcasted_iota(jnp.int32, (B, 64), 1)

    emb_hl = _split(emb_ref[...])
    winit_hl = [_split(winit_ref[b]) for b in range(2)]
    wo1_hl = [_split(wo1_ref[b]) for b in range(2)]
    wo2_hl = [_split(wo2_ref[b]) for b in range(2)]

    for ff in range(FPB):
        xd = xd_ref[ff]
        x0 = xd[:, 0:1]
        x1 = xd[:, 1:2]
        x2 = xd[:, 2:3]
        dc = jnp.sqrt((x0 * x0 + x1 * x1) + x2 * x2)
        mask = jnp.where(dc < _CUTOFF, 1.0, 0.0)

        d = jax.lax.broadcast_in_dim(dc, (P, G), (0, 1))
        arg = d - centers
        rbf = jnp.exp(arg * arg * (-0.5 / _VARIANCE))

        z1 = _bdot(rbf, wf1_ref[...])
        a1 = _softplus(z1 + bias_ref[0:1, :])
        filt = _bdot(a1, wf2_ref[...])
        filt = (filt + bias_ref[1:2, :]) * mask

        ep = ep_ref[ff]
        onehot = jnp.where(ep == vocab, 1.0, 0.0)
        oh = onehot.astype(jnp.bfloat16)
        feat = (jnp.dot(oh, emb_hl[0], preferred_element_type=jnp.float32)
                + jnp.dot(oh, emb_hl[1], preferred_element_type=jnp.float32))

        fsum = _bdot(M, filt)

        for b in range(2):
            h = _pdot(feat, winit_hl[b])
            fb = filt[:, b * FEAT:(b + 1) * FEAT]
            hsum = _bdot(S, h)
            t1 = _bdot(M, fb * hsum)
            agg = t1 - h * fsum[:, b * FEAT:(b + 1) * FEAT]
            t = _pdot(agg, wo1_hl[b])
            t = _softplus(t + bias_ref[2 + 2 * b:3 + 2 * b, :FEAT])
            out = _pdot(t, wo2_hl[b])
            out = out + bias_ref[3 + 2 * b:4 + 2 * b, :FEAT]
            feat = feat + out

        out_ref[ff] = feat


def kernel(in_features, embedding_property, emb_table, W_init, W_f1, b_f1,
           W_f2, b_f2, W_o1, b_o1, W_o2, b_o2):
    Fr, B, _ = in_features.shape
    N_EMB, FEAT = emb_table.shape
    G = 64

    pairs = np.asarray(
        [(i, j) for i in range(B) for j in range(i + 1, B)], dtype=np.int32)
    NP_REAL = pairs.shape[0]
    P = -(-NP_REAL // 256) * 256

    M_np = np.zeros((B, P), dtype=np.float32)
    M_np[pairs[:, 0], np.arange(NP_REAL)] = 1.0
    M_np[pairs[:, 1], np.arange(NP_REAL)] = 1.0
    M = jnp.asarray(M_np, dtype=jnp.bfloat16)
    S = jnp.asarray(M_np.T.copy(), dtype=jnp.bfloat16)

    x = in_features
    ia = np.zeros(P, dtype=np.int32)
    ib = np.zeros(P, dtype=np.int32)
    ia[:NP_REAL] = pairs[:, 0]
    ib[:NP_REAL] = pairs[:, 1]
    XD = (jnp.take(x, jnp.asarray(ia), axis=1)
          - jnp.take(x, jnp.asarray(ib), axis=1))
    ep3 = embedding_property.astype(jnp.int32).reshape(Fr, B, 1)
    emb_pad = jnp.pad(emb_table, ((0, 64 - N_EMB), (0, 0)))

    wf1p = jnp.pad(W_f1, ((0, 0), (0, G - _N_GAUSS), (0, 0)))
    W_f1c = jnp.concatenate([wf1p[0], wf1p[1]], axis=1)
    W_f2c = jnp.zeros((2 * FEAT, 2 * FEAT), jnp.float32)
    W_f2c = W_f2c.at[:FEAT, :FEAT].set(W_f2[0]).at[FEAT:, FEAT:].set(W_f2[1])

    def pad128(v):
        return jnp.pad(v, (0, 2 * FEAT - v.shape[0]))

    b_f2c = (jnp.concatenate([b_f2[0], b_f2[1]])
             - _LOG2 * jnp.sum(W_f2c, axis=0))
    W_f1c = W_f1c.astype(jnp.bfloat16)
    W_f2c = W_f2c.astype(jnp.bfloat16)
    bias_pack = jnp.stack([
        jnp.concatenate([b_f1[0], b_f1[1]]),
        b_f2c,
        pad128(b_o1[0]), pad128(b_o2[0] - _LOG2 * jnp.sum(W_o2[0], axis=0)),
        pad128(b_o1[1]), pad128(b_o2[1] - _LOG2 * jnp.sum(W_o2[1], axis=0)),
        jnp.zeros(2 * FEAT), jnp.zeros(2 * FEAT),
    ])

    FPB = 8
    body = functools.partial(_schnet_body, P=P, B=B, FEAT=FEAT, FPB=FPB)
    out = pl.pallas_call(
        body,
        grid=(Fr // FPB,),
        in_specs=[
            pl.BlockSpec((FPB, P, 3), lambda f: (f, 0, 0)),
            pl.BlockSpec((FPB, B, 1), lambda f: (f, 0, 0)),
            pl.BlockSpec((64, FEAT), lambda f: (0, 0)),
            pl.BlockSpec((2, FEAT, FEAT), lambda f: (0, 0, 0)),
            pl.BlockSpec((G, 2 * FEAT), lambda f: (0, 0)),
            pl.BlockSpec((2 * FEAT, 2 * FEAT), lambda f: (0, 0)),
            pl.BlockSpec((8, 2 * FEAT), lambda f: (0, 0)),
            pl.BlockSpec((2, FEAT, FEAT), lambda f: (0, 0, 0)),
            pl.BlockSpec((2, FEAT, FEAT), lambda f: (0, 0, 0)),
            pl.BlockSpec((B, P), lambda f: (0, 0)),
            pl.BlockSpec((P, B), lambda f: (0, 0)),
        ],
        out_specs=pl.BlockSpec((FPB, B, FEAT), lambda f: (f, 0, 0)),
        out_shape=jax.ShapeDtypeStruct((Fr, B, FEAT), jnp.float32),
    )(XD, ep3, emb_pad, W_init, W_f1c, W_f2c, bias_pack, W_o1, W_o2, M, S)
    return out
